# baseline (speedup 1.0000x reference)
import functools

import jax
import jax.numpy as jnp
from jax import lax
from jax.experimental import pallas as pl
from jax.experimental.pallas import tpu as pltpu
from jax.experimental.pallas import tpu_sc as plsc

N = 10000
NPAD = 10240
E = 160000
F = 256
NG = 512
NS = 64
OUT = 128

NCORE = 2
NSUB = 16
NWORK = NCORE * NSUB

GC = 200
EPW = E // NWORK


@functools.cache
def _make_sc_gather():
    @functools.partial(
        pl.kernel,
        out_type=jax.ShapeDtypeStruct((E, F), jnp.float32),
        mesh=plsc.VectorSubcoreMesh(core_axis_name="c", subcore_axis_name="s"),
        scratch_types=[
            pltpu.VMEM((GC,), jnp.int32),
            pltpu.VMEM((GC, F), jnp.float32),
            pltpu.SemaphoreType.DMA,
        ],
    )
    def _sc_gather(h_hbm, src_hbm, out_hbm, idx_v, rows_v, sem):
        wid = lax.axis_index("s") * NCORE + lax.axis_index("c")
        base0 = wid * EPW

        def step(t, carry):
            base = base0 + t * GC
            pltpu.sync_copy(src_hbm.at[pl.ds(base, GC)], idx_v)
            pltpu.async_copy(h_hbm.at[idx_v], rows_v, sem).wait()
            pltpu.sync_copy(rows_v, out_hbm.at[pl.ds(base, GC)])
            return carry

        lax.fori_loop(0, EPW // GC, step, 0)

    return _sc_gather


FS = 16
NH = NPAD // 2
ACC_C = NH + 128
DUMMY = NH + 64
SC_C = 1280
NCHUNK = E // SC_C


@functools.cache
def _make_sc_scatter():
    @functools.partial(
        pl.kernel,
        out_type=jax.ShapeDtypeStruct((F, NPAD), jnp.float32),
        mesh=plsc.VectorSubcoreMesh(core_axis_name="c", subcore_axis_name="s"),
        compiler_params=pltpu.CompilerParams(needs_layout_passes=False),
        scratch_types=[
            pltpu.VMEM((SC_C,), jnp.int32),
            pltpu.VMEM((SC_C,), jnp.int32),
            pltpu.VMEM((FS, SC_C), jnp.float32),
            pltpu.VMEM((FS, SC_C), jnp.float32),
            pltpu.SemaphoreType.DMA,
            pltpu.SemaphoreType.DMA,
            pltpu.VMEM((FS * ACC_C,), jnp.float32),
        ],
    )
    def _sc_scatter(msg_hbm, dst_hbm, out_hbm, dst0, dst1, buf0, buf1,
                    sem0, sem1, acc_v):
        c = lax.axis_index("c")
        s = lax.axis_index("s")
        base = c * NH
        iota = lax.iota(jnp.int32, 16)
        rot = [(iota + k) & 15 for k in range(16)]
        rot_acc = [((iota + k) & 15) * ACC_C for k in range(16)]
        bufs = (buf0, buf1)
        dsts = (dst0, dst1)
        sems = (sem0, sem1)

        def start(pp, t):
            pltpu.async_copy(dst_hbm.at[pl.ds(t * SC_C, SC_C)], dsts[pp],
                             sems[pp])
            pltpu.async_copy(msg_hbm.at[s, t], bufs[pp], sems[pp])

        def wait(pp, t):
            pltpu.make_async_copy(dst_hbm.at[pl.ds(t * SC_C, SC_C)],
                                  dsts[pp], sems[pp]).wait()
            pltpu.make_async_copy(msg_hbm.at[s, t], bufs[pp],
                                  sems[pp]).wait()

        def compute(pp):
            buf_v = bufs[pp]
            dst_v = dsts[pp]

            @plsc.parallel_loop(0, SC_C // 16, unroll=4)
            def group(g):
                dvec = dst_v[pl.ds(g * 16, 16)]
                l = dvec - base
                ok = (l >= 0) & (l < NH)
                lsel = jnp.where(ok, l, DUMMY)
                cidx = g * 16 + iota
                for k in range(16):
                    v = plsc.load_gather(buf_v, [rot[k], cidx])
                    plsc.addupdate_scatter(acc_v, [lsel + rot_acc[k]], v)

        start(0, 0)

        @plsc.parallel_loop(0, FS * ACC_C // 16, unroll=4)
        def zrow(i):
            acc_v[pl.ds(i * 16, 16)] = jnp.zeros((16,), jnp.float32)

        def pair(u, carry):
            start(1, 2 * u + 1)
            wait(0, 2 * u)
            compute(0)
            start(0, 2 * u + 2)
            wait(1, 2 * u + 1)
            compute(1)
            return carry

        lax.fori_loop(0, (NCHUNK - 1) // 2, pair, 0)
        wait(0, NCHUNK - 1)
        compute(0)

        for cc in range(16):
            pltpu.sync_copy(acc_v.at[pl.ds(cc * ACC_C, NH)],
                            out_hbm.at[FS * s + cc, pl.ds(base, NH)])

    return _sc_scatter


BE = 1280


def _msg_body(ea_ref, hs_ref, ab_ref, w1_ref, b1_ref, w2_ref, b2_ref, o_ref):
    e = jnp.dot(ea_ref[...], w1_ref[...], preferred_element_type=jnp.float32)
    e = jnp.maximum(e + b1_ref[...], 0.0)
    e = jnp.dot(e, w2_ref[...], preferred_element_type=jnp.float32) + b2_ref[...]
    hs = hs_ref[...] * ab_ref[0:1, :] + ab_ref[1:2, :]
    m = jnp.maximum(e + hs, 0.0)
    o_ref[...] = jnp.transpose(m).reshape(FS, 1, FS, BE)


_msg_call = pl.pallas_call(
    _msg_body,
    grid=(E // BE,),
    in_specs=[
        pl.BlockSpec((BE, 16), lambda i: (i, 0)),
        pl.BlockSpec((BE, F), lambda i: (i, 0)),
        pl.BlockSpec((8, F), lambda i: (0, 0)),
        pl.BlockSpec((16, F), lambda i: (0, 0)),
        pl.BlockSpec((1, F), lambda i: (0, 0)),
        pl.BlockSpec((F, F), lambda i: (0, 0)),
        pl.BlockSpec((1, F), lambda i: (0, 0)),
    ],
    out_specs=pl.BlockSpec((FS, 1, FS, BE), lambda i: (0, i, 0, 0)),
    out_shape=jax.ShapeDtypeStruct((FS, E // BE, FS, BE), jnp.float32),
)

BN = 1280
_NBLK = NPAD // BN


def _node_body(y_ref, ab_ref, aggT_ref, eps_ref, w1_ref, b1_ref, w2_ref,
               b2_ref, g_ref, bb_ref, yo_ref, abo_ref, acc):
    i = pl.program_id(0)

    @pl.when(i == 0)
    def _():
        acc[...] = jnp.zeros_like(acc)

    x = y_ref[...] * ab_ref[0:1, :] + ab_ref[1:2, :]
    agg = jnp.transpose(aggT_ref[...])
    o = (1.0 + eps_ref[0, 0]) * x + agg
    t = jnp.dot(o, w1_ref[...], preferred_element_type=jnp.float32)
    t = jnp.maximum(t + b1_ref[...], 0.0)
    y = jnp.dot(t, w2_ref[...], preferred_element_type=jnp.float32)
    y = jnp.maximum(y + b2_ref[...], 0.0)
    yo_ref[...] = y
    gidx = i * BN + lax.broadcasted_iota(jnp.int32, (BN, 1), 0)
    ym = y * (gidx < N).astype(jnp.float32)
    acc[0:1, :] += jnp.sum(ym, axis=0, keepdims=True)
    acc[1:2, :] += jnp.sum(ym * ym, axis=0, keepdims=True)

    @pl.when(i == _NBLK - 1)
    def _():
        mu = acc[0:1, :] * (1.0 / N)
        var = acc[1:2, :] * (1.0 / N) - mu * mu
        al = g_ref[...] * lax.rsqrt(var + 1e-5)
        be = bb_ref[...] - mu * al
        abo_ref[...] = jnp.concatenate(
            [al, be, jnp.zeros((6, F), jnp.float32)], axis=0)


_node_call = pl.pallas_call(
    _node_body,
    grid=(_NBLK,),
    in_specs=[
        pl.BlockSpec((BN, F), lambda i: (i, 0)),
        pl.BlockSpec((8, F), lambda i: (0, 0)),
        pl.BlockSpec((F, BN), lambda i: (0, i)),
        pl.BlockSpec((1, 1), lambda i: (0, 0)),
        pl.BlockSpec((F, F), lambda i: (0, 0)),
        pl.BlockSpec((1, F), lambda i: (0, 0)),
        pl.BlockSpec((F, F), lambda i: (0, 0)),
        pl.BlockSpec((1, F), lambda i: (0, 0)),
        pl.BlockSpec((1, F), lambda i: (0, 0)),
        pl.BlockSpec((1, F), lambda i: (0, 0)),
    ],
    out_specs=[
        pl.BlockSpec((BN, F), lambda i: (i, 0)),
        pl.BlockSpec((8, F), lambda i: (0, 0)),
    ],
    out_shape=[
        jax.ShapeDtypeStruct((NPAD, F), jnp.float32),
        jax.ShapeDtypeStruct((8, F), jnp.float32),
    ],
    scratch_shapes=[pltpu.VMEM((8, F), jnp.float32)],
)

PB = 1280
_PBLK = NPAD // PB


def _pool_body(y1, y2, y3, y4, a1, a2, a3, a4, bt_ref, w_ref, sg_ref,
               f7w, f7b, pw, pb, out_ref, s_acc, c_acc):
    i = pl.program_id(0)

    @pl.when(i == 0)
    def _():
        s_acc[...] = jnp.zeros_like(s_acc)
        c_acc[...] = jnp.zeros_like(c_acc)

    bb = bt_ref[0, :, :]
    iota = lax.broadcasted_iota(jnp.int32, (NG, PB), 0)
    mask = (iota == bb).astype(jnp.float32)
    xc = jnp.concatenate([
        y1[...] * a1[0:1, :] + a1[1:2, :],
        y2[...] * a2[0:1, :] + a2[1:2, :],
        y3[...] * a3[0:1, :] + a3[1:2, :],
        y4[...] * a4[0:1, :] + a4[1:2, :],
    ], axis=1)
    s_acc[...] += jnp.dot(mask, xc, preferred_element_type=jnp.float32)
    c_acc[:, 0:1] += jnp.sum(mask, axis=1, keepdims=True)

    @pl.when(i == _PBLK - 1)
    def _():
        cnt = c_acc[:, 0:1]
        g = s_acc[...] / jnp.maximum(cnt, 1.0) * w_ref[...]
        sg = sg_ref[0:1, :]
        iota2 = lax.broadcasted_iota(jnp.int32, (NS, NG), 0)
        m2 = (iota2 == sg).astype(jnp.float32)
        gs = jnp.dot(m2, g, preferred_element_type=jnp.float32)
        norm = jnp.dot(m2, w_ref[...], preferred_element_type=jnp.float32)
        gs = gs / jnp.where(norm == 0.0, 1.0, norm)
        h = jnp.dot(gs, f7w[...], preferred_element_type=jnp.float32)
        h = jnp.maximum(h + f7b[...], 0.0)
        out_ref[...] = jnp.dot(h, pw[...],
                               preferred_element_type=jnp.float32) + pb[...]


_pool_call = pl.pallas_call(
    _pool_body,
    grid=(_PBLK,),
    in_specs=(
        [pl.BlockSpec((PB, F), lambda i: (i, 0)) for _ in range(4)]
        + [pl.BlockSpec((8, F), lambda i: (0, 0)) for _ in range(4)]
        + [
            pl.BlockSpec((1, 1, PB), lambda i: (i, 0, 0)),
            pl.BlockSpec((NG, 1), lambda i: (0, 0)),
            pl.BlockSpec((1, NG), lambda i: (0, 0)),
            pl.BlockSpec((4 * F, F), lambda i: (0, 0)),
            pl.BlockSpec((1, F), lambda i: (0, 0)),
            pl.BlockSpec((F, OUT), lambda i: (0, 0)),
            pl.BlockSpec((1, OUT), lambda i: (0, 0)),
        ]
    ),
    out_specs=pl.BlockSpec((NS, OUT), lambda i: (0, 0)),
    out_shape=jax.ShapeDtypeStruct((NS, OUT), jnp.float32),
    scratch_shapes=[
        pltpu.VMEM((NG, 4 * F), jnp.float32),
        pltpu.VMEM((NG, 128), jnp.float32),
    ],
)


def kernel(x, edge_index, edge_attr, batch, weights, subgraph_batch, params):
    p = params
    src = edge_index[0]
    dst = edge_index[1]
    xpad = jnp.concatenate(
        [x, jnp.zeros((NPAD - N, F), jnp.float32)], axis=0)
    batchp = jnp.concatenate(
        [batch, jnp.full((NPAD - N,), NG, jnp.int32)])
    batch3 = batchp.reshape(_PBLK, 1, PB)
    sgb2 = subgraph_batch.reshape(1, NG)

    sc_gather = _make_sc_gather()
    sc_scatter = _make_sc_scatter()
    ab = jnp.concatenate(
        [jnp.ones((1, F), jnp.float32), jnp.zeros((7, F), jnp.float32)],
        axis=0)
    y = xpad
    ys, abs_ = [], []
    for i in (1, 2, 3, 4):
        hs = sc_gather(y, src)
        msgT = _msg_call(edge_attr, hs, ab,
                         p['c%d_beW1' % i], p['c%d_beb1' % i].reshape(1, F),
                         p['c%d_beW2' % i], p['c%d_beb2' % i].reshape(1, F))
        aggT = sc_scatter(msgT, dst)
        y, ab = _node_call(y, ab, aggT,
                           p['c%d_eps' % i].reshape(1, 1),
                           p['c%d_mW1' % i], p['c%d_mb1' % i].reshape(1, F),
                           p['c%d_mW2' % i], p['c%d_mb2' % i].reshape(1, F),
                           p['bn%d_g' % i].reshape(1, F),
                           p['bn%d_b' % i].reshape(1, F))
        ys.append(y)
        abs_.append(ab)

    return _pool_call(*ys, *abs_, batch3, weights, sgb2,
                      p['fc7_W'], p['fc7_b'].reshape(1, F),
                      p['pred_W'], p['pred_b'].reshape(1, OUT))


# speedup vs baseline: 2.1083x; 2.1083x over previous
import functools

import jax
import jax.numpy as jnp
from jax import lax
from jax.experimental import pallas as pl
from jax.experimental.pallas import tpu as pltpu
from jax.experimental.pallas import tpu_sc as plsc

N = 10000
NPAD = 10240
E = 160000
F = 256
NG = 512
NS = 64
OUT = 128

NCORE = 2
NSUB = 16
NWORK = NCORE * NSUB

GC = 200
EPW = E // NWORK


@functools.cache
def _make_sc_gather():
    @functools.partial(
        pl.kernel,
        out_type=jax.ShapeDtypeStruct((E, F), jnp.float32),
        mesh=plsc.VectorSubcoreMesh(core_axis_name="c", subcore_axis_name="s"),
        scratch_types=[
            pltpu.VMEM((GC,), jnp.int32),
            pltpu.VMEM((GC, F), jnp.float32),
            pltpu.SemaphoreType.DMA,
        ],
    )
    def _sc_gather(h_hbm, src_hbm, out_hbm, idx_v, rows_v, sem):
        wid = lax.axis_index("s") * NCORE + lax.axis_index("c")
        base0 = wid * EPW

        def step(t, carry):
            base = base0 + t * GC
            pltpu.sync_copy(src_hbm.at[pl.ds(base, GC)], idx_v)
            pltpu.async_copy(h_hbm.at[idx_v], rows_v, sem).wait()
            pltpu.sync_copy(rows_v, out_hbm.at[pl.ds(base, GC)])
            return carry

        lax.fori_loop(0, EPW // GC, step, 0)

    return _sc_gather


FS = 16
NH = NPAD // 2
ACC_C = NH + 128
DUMMY = NH + 64
SC_C = 1280
NCHUNK = E // SC_C


@functools.cache
def _make_sc_scatter():
    @functools.partial(
        pl.kernel,
        out_type=jax.ShapeDtypeStruct((F, NPAD), jnp.float32),
        mesh=plsc.VectorSubcoreMesh(core_axis_name="c", subcore_axis_name="s"),
        compiler_params=pltpu.CompilerParams(needs_layout_passes=False),
        scratch_types=[
            pltpu.VMEM((SC_C,), jnp.int32),
            pltpu.VMEM((SC_C,), jnp.int32),
            pltpu.VMEM((FS, SC_C), jnp.float32),
            pltpu.VMEM((FS, SC_C), jnp.float32),
            pltpu.SemaphoreType.DMA,
            pltpu.SemaphoreType.DMA,
            pltpu.VMEM((FS * ACC_C,), jnp.float32),
        ],
    )
    def _sc_scatter(msg_hbm, dst_hbm, out_hbm, dst0, dst1, buf0, buf1,
                    sem0, sem1, acc_v):
        c = lax.axis_index("c")
        s = lax.axis_index("s")
        base = c * NH
        iota = lax.iota(jnp.int32, 16)
        rot = [(iota + k) & 15 for k in range(16)]
        bufs = (buf0, buf1)
        dsts = (dst0, dst1)
        sems = (sem0, sem1)

        def start(pp, t):
            pltpu.async_copy(dst_hbm.at[pl.ds(t * SC_C, SC_C)], dsts[pp],
                             sems[pp])
            pltpu.async_copy(msg_hbm.at[s, t], bufs[pp], sems[pp])

        def wait(pp, t):
            pltpu.make_async_copy(dst_hbm.at[pl.ds(t * SC_C, SC_C)],
                                  dsts[pp], sems[pp]).wait()
            pltpu.make_async_copy(msg_hbm.at[s, t], bufs[pp],
                                  sems[pp]).wait()

        def compute(pp):
            buf_v = bufs[pp]
            dst_v = dsts[pp]

            @plsc.parallel_loop(0, SC_C // 16, unroll=2)
            def group(g):
                dvec = dst_v[pl.ds(g * 16, 16)]
                l = dvec - base
                ok = (l >= 0) & (l < NH)
                fbase = jnp.where(ok, l, DUMMY) * 16
                cidx = g * 16 + iota
                for k in range(16):
                    v = plsc.load_gather(buf_v, [rot[k], cidx])
                    plsc.addupdate_scatter(acc_v, [fbase + rot[k]], v)

        start(0, 0)

        @plsc.parallel_loop(0, FS * ACC_C // 16, unroll=4)
        def zrow(i):
            acc_v[pl.ds(i * 16, 16)] = jnp.zeros((16,), jnp.float32)

        def pair(u, carry):
            start(1, 2 * u + 1)
            wait(0, 2 * u)
            compute(0)
            start(0, 2 * u + 2)
            wait(1, 2 * u + 1)
            compute(1)
            return carry

        lax.fori_loop(0, (NCHUNK - 1) // 2, pair, 0)
        wait(0, NCHUNK - 1)
        compute(0)

        for q in range(NH // SC_C):
            @plsc.parallel_loop(0, SC_C // 16, unroll=2)
            def tgroup(g):
                nbase = (q * SC_C + g * 16) * 16 + iota * 16
                cidx = g * 16 + iota
                for k in range(16):
                    v = plsc.load_gather(acc_v, [nbase + rot[k]])
                    plsc.store_scatter(buf0, [rot[k], cidx], v)

            pltpu.sync_copy(
                buf0, out_hbm.at[pl.ds(FS * s, FS),
                                 pl.ds(base + q * SC_C, SC_C)])

    return _sc_scatter


BE = 1280


def _msg_body(ea_ref, hs_ref, ab_ref, w1_ref, b1_ref, w2_ref, b2_ref, o_ref):
    e = jnp.dot(ea_ref[...], w1_ref[...], preferred_element_type=jnp.float32)
    e = jnp.maximum(e + b1_ref[...], 0.0)
    e = jnp.dot(e, w2_ref[...], preferred_element_type=jnp.float32) + b2_ref[...]
    hs = hs_ref[...] * ab_ref[0:1, :] + ab_ref[1:2, :]
    m = jnp.maximum(e + hs, 0.0)
    o_ref[...] = jnp.transpose(m).reshape(FS, 1, FS, BE)


_msg_call = pl.pallas_call(
    _msg_body,
    grid=(E // BE,),
    in_specs=[
        pl.BlockSpec((BE, 16), lambda i: (i, 0)),
        pl.BlockSpec((BE, F), lambda i: (i, 0)),
        pl.BlockSpec((8, F), lambda i: (0, 0)),
        pl.BlockSpec((16, F), lambda i: (0, 0)),
        pl.BlockSpec((1, F), lambda i: (0, 0)),
        pl.BlockSpec((F, F), lambda i: (0, 0)),
        pl.BlockSpec((1, F), lambda i: (0, 0)),
    ],
    out_specs=pl.BlockSpec((FS, 1, FS, BE), lambda i: (0, i, 0, 0)),
    out_shape=jax.ShapeDtypeStruct((FS, E // BE, FS, BE), jnp.float32),
)

BN = 1280
_NBLK = NPAD // BN


def _node_body(y_ref, ab_ref, aggT_ref, eps_ref, w1_ref, b1_ref, w2_ref,
               b2_ref, g_ref, bb_ref, yo_ref, abo_ref, acc):
    i = pl.program_id(0)

    @pl.when(i == 0)
    def _():
        acc[...] = jnp.zeros_like(acc)

    x = y_ref[...] * ab_ref[0:1, :] + ab_ref[1:2, :]
    agg = jnp.transpose(aggT_ref[...])
    o = (1.0 + eps_ref[0, 0]) * x + agg
    t = jnp.dot(o, w1_ref[...], preferred_element_type=jnp.float32)
    t = jnp.maximum(t + b1_ref[...], 0.0)
    y = jnp.dot(t, w2_ref[...], preferred_element_type=jnp.float32)
    y = jnp.maximum(y + b2_ref[...], 0.0)
    yo_ref[...] = y
    gidx = i * BN + lax.broadcasted_iota(jnp.int32, (BN, 1), 0)
    ym = y * (gidx < N).astype(jnp.float32)
    acc[0:1, :] += jnp.sum(ym, axis=0, keepdims=True)
    acc[1:2, :] += jnp.sum(ym * ym, axis=0, keepdims=True)

    @pl.when(i == _NBLK - 1)
    def _():
        mu = acc[0:1, :] * (1.0 / N)
        var = acc[1:2, :] * (1.0 / N) - mu * mu
        al = g_ref[...] * lax.rsqrt(var + 1e-5)
        be = bb_ref[...] - mu * al
        abo_ref[...] = jnp.concatenate(
            [al, be, jnp.zeros((6, F), jnp.float32)], axis=0)


_node_call = pl.pallas_call(
    _node_body,
    grid=(_NBLK,),
    in_specs=[
        pl.BlockSpec((BN, F), lambda i: (i, 0)),
        pl.BlockSpec((8, F), lambda i: (0, 0)),
        pl.BlockSpec((F, BN), lambda i: (0, i)),
        pl.BlockSpec((1, 1), lambda i: (0, 0)),
        pl.BlockSpec((F, F), lambda i: (0, 0)),
        pl.BlockSpec((1, F), lambda i: (0, 0)),
        pl.BlockSpec((F, F), lambda i: (0, 0)),
        pl.BlockSpec((1, F), lambda i: (0, 0)),
        pl.BlockSpec((1, F), lambda i: (0, 0)),
        pl.BlockSpec((1, F), lambda i: (0, 0)),
    ],
    out_specs=[
        pl.BlockSpec((BN, F), lambda i: (i, 0)),
        pl.BlockSpec((8, F), lambda i: (0, 0)),
    ],
    out_shape=[
        jax.ShapeDtypeStruct((NPAD, F), jnp.float32),
        jax.ShapeDtypeStruct((8, F), jnp.float32),
    ],
    scratch_shapes=[pltpu.VMEM((8, F), jnp.float32)],
)

PB = 1280
_PBLK = NPAD // PB


def _pool_body(y1, y2, y3, y4, a1, a2, a3, a4, bt_ref, w_ref, sg_ref,
               f7w, f7b, pw, pb, out_ref, s_acc, c_acc):
    i = pl.program_id(0)

    @pl.when(i == 0)
    def _():
        s_acc[...] = jnp.zeros_like(s_acc)
        c_acc[...] = jnp.zeros_like(c_acc)

    bb = bt_ref[0, :, :]
    iota = lax.broadcasted_iota(jnp.int32, (NG, PB), 0)
    mask = (iota == bb).astype(jnp.float32)
    xc = jnp.concatenate([
        y1[...] * a1[0:1, :] + a1[1:2, :],
        y2[...] * a2[0:1, :] + a2[1:2, :],
        y3[...] * a3[0:1, :] + a3[1:2, :],
        y4[...] * a4[0:1, :] + a4[1:2, :],
    ], axis=1)
    s_acc[...] += jnp.dot(mask, xc, preferred_element_type=jnp.float32)
    c_acc[:, 0:1] += jnp.sum(mask, axis=1, keepdims=True)

    @pl.when(i == _PBLK - 1)
    def _():
        cnt = c_acc[:, 0:1]
        g = s_acc[...] / jnp.maximum(cnt, 1.0) * w_ref[...]
        sg = sg_ref[0:1, :]
        iota2 = lax.broadcasted_iota(jnp.int32, (NS, NG), 0)
        m2 = (iota2 == sg).astype(jnp.float32)
        gs = jnp.dot(m2, g, preferred_element_type=jnp.float32)
        norm = jnp.dot(m2, w_ref[...], preferred_element_type=jnp.float32)
        gs = gs / jnp.where(norm == 0.0, 1.0, norm)
        h = jnp.dot(gs, f7w[...], preferred_element_type=jnp.float32)
        h = jnp.maximum(h + f7b[...], 0.0)
        out_ref[...] = jnp.dot(h, pw[...],
                               preferred_element_type=jnp.float32) + pb[...]


_pool_call = pl.pallas_call(
    _pool_body,
    grid=(_PBLK,),
    in_specs=(
        [pl.BlockSpec((PB, F), lambda i: (i, 0)) for _ in range(4)]
        + [pl.BlockSpec((8, F), lambda i: (0, 0)) for _ in range(4)]
        + [
            pl.BlockSpec((1, 1, PB), lambda i: (i, 0, 0)),
            pl.BlockSpec((NG, 1), lambda i: (0, 0)),
            pl.BlockSpec((1, NG), lambda i: (0, 0)),
            pl.BlockSpec((4 * F, F), lambda i: (0, 0)),
            pl.BlockSpec((1, F), lambda i: (0, 0)),
            pl.BlockSpec((F, OUT), lambda i: (0, 0)),
            pl.BlockSpec((1, OUT), lambda i: (0, 0)),
        ]
    ),
    out_specs=pl.BlockSpec((NS, OUT), lambda i: (0, 0)),
    out_shape=jax.ShapeDtypeStruct((NS, OUT), jnp.float32),
    scratch_shapes=[
        pltpu.VMEM((NG, 4 * F), jnp.float32),
        pltpu.VMEM((NG, 128), jnp.float32),
    ],
)


def kernel(x, edge_index, edge_attr, batch, weights, subgraph_batch, params):
    p = params
    src = edge_index[0]
    dst = edge_index[1]
    xpad = jnp.concatenate(
        [x, jnp.zeros((NPAD - N, F), jnp.float32)], axis=0)
    batchp = jnp.concatenate(
        [batch, jnp.full((NPAD - N,), NG, jnp.int32)])
    batch3 = batchp.reshape(_PBLK, 1, PB)
    sgb2 = subgraph_batch.reshape(1, NG)

    sc_gather = _make_sc_gather()
    sc_scatter = _make_sc_scatter()
    ab = jnp.concatenate(
        [jnp.ones((1, F), jnp.float32), jnp.zeros((7, F), jnp.float32)],
        axis=0)
    y = xpad
    ys, abs_ = [], []
    for i in (1, 2, 3, 4):
        hs = sc_gather(y, src)
        msgT = _msg_call(edge_attr, hs, ab,
                         p['c%d_beW1' % i], p['c%d_beb1' % i].reshape(1, F),
                         p['c%d_beW2' % i], p['c%d_beb2' % i].reshape(1, F))
        aggT = sc_scatter(msgT, dst)
        y, ab = _node_call(y, ab, aggT,
                           p['c%d_eps' % i].reshape(1, 1),
                           p['c%d_mW1' % i], p['c%d_mb1' % i].reshape(1, F),
                           p['c%d_mW2' % i], p['c%d_mb2' % i].reshape(1, F),
                           p['bn%d_g' % i].reshape(1, F),
                           p['bn%d_b' % i].reshape(1, F))
        ys.append(y)
        abs_.append(ab)

    return _pool_call(*ys, *abs_, batch3, weights, sgb2,
                      p['fc7_W'], p['fc7_b'].reshape(1, F),
                      p['pred_W'], p['pred_b'].reshape(1, OUT))


# speedup vs baseline: 2.1819x; 1.0349x over previous
import functools

import jax
import jax.numpy as jnp
from jax import lax
from jax.experimental import pallas as pl
from jax.experimental.pallas import tpu as pltpu
from jax.experimental.pallas import tpu_sc as plsc

N = 10000
NPAD = 10240
E = 160000
F = 256
NG = 512
NS = 64
OUT = 128

NCORE = 2
NSUB = 16
NWORK = NCORE * NSUB

GC = 200
EPW = E // NWORK


@functools.cache
def _make_sc_gather():
    @functools.partial(
        pl.kernel,
        out_type=jax.ShapeDtypeStruct((E, F), jnp.float32),
        mesh=plsc.VectorSubcoreMesh(core_axis_name="c", subcore_axis_name="s"),
        scratch_types=[
            pltpu.VMEM((GC,), jnp.int32),
            pltpu.VMEM((GC,), jnp.int32),
            pltpu.VMEM((GC, F), jnp.float32),
            pltpu.VMEM((GC, F), jnp.float32),
            pltpu.SemaphoreType.DMA,
            pltpu.SemaphoreType.DMA,
            pltpu.SemaphoreType.DMA,
            pltpu.SemaphoreType.DMA,
        ],
    )
    def _sc_gather(h_hbm, src_hbm, out_hbm, idx0, idx1, rows0, rows1,
                   si0, si1, sg0, sg1):
        wid = lax.axis_index("s") * NCORE + lax.axis_index("c")
        base0 = wid * EPW
        idxs = (idx0, idx1)
        rows = (rows0, rows1)
        sis = (si0, si1)
        sgs = (sg0, sg1)

        def istart(pp, t):
            pltpu.async_copy(src_hbm.at[pl.ds(base0 + t * GC, GC)],
                             idxs[pp], sis[pp])

        def iwait(pp, t):
            pltpu.make_async_copy(src_hbm.at[pl.ds(base0 + t * GC, GC)],
                                  idxs[pp], sis[pp]).wait()

        def gstart(pp):
            pltpu.async_copy(h_hbm.at[idxs[pp]], rows[pp], sgs[pp])

        def gwait(pp):
            pltpu.make_async_copy(h_hbm.at[idxs[pp]], rows[pp],
                                  sgs[pp]).wait()

        def wstart(pp, t):
            pltpu.async_copy(rows[pp], out_hbm.at[pl.ds(base0 + t * GC, GC)],
                             sgs[pp])

        def wwait(pp, t):
            pltpu.make_async_copy(rows[pp],
                                  out_hbm.at[pl.ds(base0 + t * GC, GC)],
                                  sgs[pp]).wait()

        istart(0, 0)
        iwait(0, 0)
        gstart(0)

        def pair(u, carry):
            t0 = 2 * u
            istart(1, t0 + 1)
            gwait(0)
            wstart(0, t0)
            iwait(1, t0 + 1)
            gstart(1)

            @pl.when(t0 + 2 < EPW // GC)
            def _():
                istart(0, t0 + 2)
                wwait(0, t0)
                iwait(0, t0 + 2)
                gstart(0)

            gwait(1)
            wstart(1, t0 + 1)
            wwait(1, t0 + 1)
            return carry

        lax.fori_loop(0, EPW // GC // 2, pair, 0)
        gwait(0)
        wstart(0, EPW // GC - 1)
        wwait(0, EPW // GC - 1)

    return _sc_gather


FS = 16
NH = NPAD // 2
ACC_C = NH + 128
DUMMY = NH + 64
SC_C = 1280
NCHUNK = E // SC_C


@functools.cache
def _make_sc_scatter():
    @functools.partial(
        pl.kernel,
        out_type=jax.ShapeDtypeStruct((F, NPAD), jnp.float32),
        mesh=plsc.VectorSubcoreMesh(core_axis_name="c", subcore_axis_name="s"),
        compiler_params=pltpu.CompilerParams(needs_layout_passes=False),
        scratch_types=[
            pltpu.VMEM((SC_C,), jnp.int32),
            pltpu.VMEM((SC_C,), jnp.int32),
            pltpu.VMEM((FS, SC_C), jnp.float32),
            pltpu.VMEM((FS, SC_C), jnp.float32),
            pltpu.SemaphoreType.DMA,
            pltpu.SemaphoreType.DMA,
            pltpu.VMEM((FS * ACC_C,), jnp.float32),
        ],
    )
    def _sc_scatter(msg_hbm, dst_hbm, out_hbm, dst0, dst1, buf0, buf1,
                    sem0, sem1, acc_v):
        c = lax.axis_index("c")
        s = lax.axis_index("s")
        base = c * NH
        iota = lax.iota(jnp.int32, 16)
        rot = [(iota + k) & 15 for k in range(16)]
        bufs = (buf0, buf1)
        dsts = (dst0, dst1)
        sems = (sem0, sem1)

        def start(pp, t):
            pltpu.async_copy(dst_hbm.at[pl.ds(t * SC_C, SC_C)], dsts[pp],
                             sems[pp])
            pltpu.async_copy(msg_hbm.at[s, t], bufs[pp], sems[pp])

        def wait(pp, t):
            pltpu.make_async_copy(dst_hbm.at[pl.ds(t * SC_C, SC_C)],
                                  dsts[pp], sems[pp]).wait()
            pltpu.make_async_copy(msg_hbm.at[s, t], bufs[pp],
                                  sems[pp]).wait()

        def compute(pp):
            buf_v = bufs[pp]
            dst_v = dsts[pp]

            @plsc.parallel_loop(0, SC_C // 16, unroll=2)
            def group(g):
                dvec = dst_v[pl.ds(g * 16, 16)]
                l = dvec - base
                ok = (l >= 0) & (l < NH)
                fbase = jnp.where(ok, l, DUMMY) * 16
                cidx = g * 16 + iota
                for k in range(16):
                    v = plsc.load_gather(buf_v, [rot[k], cidx])
                    plsc.addupdate_scatter(acc_v, [fbase + rot[k]], v)

        start(0, 0)

        @plsc.parallel_loop(0, FS * ACC_C // 16, unroll=4)
        def zrow(i):
            acc_v[pl.ds(i * 16, 16)] = jnp.zeros((16,), jnp.float32)

        def pair(u, carry):
            start(1, 2 * u + 1)
            wait(0, 2 * u)
            compute(0)
            start(0, 2 * u + 2)
            wait(1, 2 * u + 1)
            compute(1)
            return carry

        lax.fori_loop(0, (NCHUNK - 1) // 2, pair, 0)
        wait(0, NCHUNK - 1)
        compute(0)

        for q in range(NH // SC_C):
            @plsc.parallel_loop(0, SC_C // 16, unroll=2)
            def tgroup(g):
                nbase = (q * SC_C + g * 16) * 16 + iota * 16
                cidx = g * 16 + iota
                for k in range(16):
                    v = plsc.load_gather(acc_v, [nbase + rot[k]])
                    plsc.store_scatter(buf0, [rot[k], cidx], v)

            pltpu.sync_copy(
                buf0, out_hbm.at[pl.ds(FS * s, FS),
                                 pl.ds(base + q * SC_C, SC_C)])

    return _sc_scatter


BE = 1280


def _msg_body(ea_ref, hs_ref, ab_ref, w1_ref, b1_ref, w2_ref, b2_ref, o_ref):
    e = jnp.dot(ea_ref[...], w1_ref[...], preferred_element_type=jnp.float32)
    e = jnp.maximum(e + b1_ref[...], 0.0)
    e = jnp.dot(e, w2_ref[...], preferred_element_type=jnp.float32) + b2_ref[...]
    hs = hs_ref[...] * ab_ref[0:1, :] + ab_ref[1:2, :]
    m = jnp.maximum(e + hs, 0.0)
    o_ref[...] = jnp.transpose(m).reshape(FS, 1, FS, BE)


_msg_call = pl.pallas_call(
    _msg_body,
    grid=(E // BE,),
    in_specs=[
        pl.BlockSpec((BE, 16), lambda i: (i, 0)),
        pl.BlockSpec((BE, F), lambda i: (i, 0)),
        pl.BlockSpec((8, F), lambda i: (0, 0)),
        pl.BlockSpec((16, F), lambda i: (0, 0)),
        pl.BlockSpec((1, F), lambda i: (0, 0)),
        pl.BlockSpec((F, F), lambda i: (0, 0)),
        pl.BlockSpec((1, F), lambda i: (0, 0)),
    ],
    out_specs=pl.BlockSpec((FS, 1, FS, BE), lambda i: (0, i, 0, 0)),
    out_shape=jax.ShapeDtypeStruct((FS, E // BE, FS, BE), jnp.float32),
)

BN = 1280
_NBLK = NPAD // BN


def _node_body(y_ref, ab_ref, aggT_ref, eps_ref, w1_ref, b1_ref, w2_ref,
               b2_ref, g_ref, bb_ref, yo_ref, abo_ref, acc):
    i = pl.program_id(0)

    @pl.when(i == 0)
    def _():
        acc[...] = jnp.zeros_like(acc)

    x = y_ref[...] * ab_ref[0:1, :] + ab_ref[1:2, :]
    agg = jnp.transpose(aggT_ref[...])
    o = (1.0 + eps_ref[0, 0]) * x + agg
    t = jnp.dot(o, w1_ref[...], preferred_element_type=jnp.float32)
    t = jnp.maximum(t + b1_ref[...], 0.0)
    y = jnp.dot(t, w2_ref[...], preferred_element_type=jnp.float32)
    y = jnp.maximum(y + b2_ref[...], 0.0)
    yo_ref[...] = y
    gidx = i * BN + lax.broadcasted_iota(jnp.int32, (BN, 1), 0)
    ym = y * (gidx < N).astype(jnp.float32)
    acc[0:1, :] += jnp.sum(ym, axis=0, keepdims=True)
    acc[1:2, :] += jnp.sum(ym * ym, axis=0, keepdims=True)

    @pl.when(i == _NBLK - 1)
    def _():
        mu = acc[0:1, :] * (1.0 / N)
        var = acc[1:2, :] * (1.0 / N) - mu * mu
        al = g_ref[...] * lax.rsqrt(var + 1e-5)
        be = bb_ref[...] - mu * al
        abo_ref[...] = jnp.concatenate(
            [al, be, jnp.zeros((6, F), jnp.float32)], axis=0)


_node_call = pl.pallas_call(
    _node_body,
    grid=(_NBLK,),
    in_specs=[
        pl.BlockSpec((BN, F), lambda i: (i, 0)),
        pl.BlockSpec((8, F), lambda i: (0, 0)),
        pl.BlockSpec((F, BN), lambda i: (0, i)),
        pl.BlockSpec((1, 1), lambda i: (0, 0)),
        pl.BlockSpec((F, F), lambda i: (0, 0)),
        pl.BlockSpec((1, F), lambda i: (0, 0)),
        pl.BlockSpec((F, F), lambda i: (0, 0)),
        pl.BlockSpec((1, F), lambda i: (0, 0)),
        pl.BlockSpec((1, F), lambda i: (0, 0)),
        pl.BlockSpec((1, F), lambda i: (0, 0)),
    ],
    out_specs=[
        pl.BlockSpec((BN, F), lambda i: (i, 0)),
        pl.BlockSpec((8, F), lambda i: (0, 0)),
    ],
    out_shape=[
        jax.ShapeDtypeStruct((NPAD, F), jnp.float32),
        jax.ShapeDtypeStruct((8, F), jnp.float32),
    ],
    scratch_shapes=[pltpu.VMEM((8, F), jnp.float32)],
)

PB = 1280
_PBLK = NPAD // PB


def _pool_body(y1, y2, y3, y4, a1, a2, a3, a4, bt_ref, w_ref, sg_ref,
               f7w, f7b, pw, pb, out_ref, s_acc, c_acc):
    i = pl.program_id(0)

    @pl.when(i == 0)
    def _():
        s_acc[...] = jnp.zeros_like(s_acc)
        c_acc[...] = jnp.zeros_like(c_acc)

    bb = bt_ref[0, :, :]
    iota = lax.broadcasted_iota(jnp.int32, (NG, PB), 0)
    mask = (iota == bb).astype(jnp.float32)
    xc = jnp.concatenate([
        y1[...] * a1[0:1, :] + a1[1:2, :],
        y2[...] * a2[0:1, :] + a2[1:2, :],
        y3[...] * a3[0:1, :] + a3[1:2, :],
        y4[...] * a4[0:1, :] + a4[1:2, :],
    ], axis=1)
    s_acc[...] += jnp.dot(mask, xc, preferred_element_type=jnp.float32)
    c_acc[:, 0:1] += jnp.sum(mask, axis=1, keepdims=True)

    @pl.when(i == _PBLK - 1)
    def _():
        cnt = c_acc[:, 0:1]
        g = s_acc[...] / jnp.maximum(cnt, 1.0) * w_ref[...]
        sg = sg_ref[0:1, :]
        iota2 = lax.broadcasted_iota(jnp.int32, (NS, NG), 0)
        m2 = (iota2 == sg).astype(jnp.float32)
        gs = jnp.dot(m2, g, preferred_element_type=jnp.float32)
        norm = jnp.dot(m2, w_ref[...], preferred_element_type=jnp.float32)
        gs = gs / jnp.where(norm == 0.0, 1.0, norm)
        h = jnp.dot(gs, f7w[...], preferred_element_type=jnp.float32)
        h = jnp.maximum(h + f7b[...], 0.0)
        out_ref[...] = jnp.dot(h, pw[...],
                               preferred_element_type=jnp.float32) + pb[...]


_pool_call = pl.pallas_call(
    _pool_body,
    grid=(_PBLK,),
    in_specs=(
        [pl.BlockSpec((PB, F), lambda i: (i, 0)) for _ in range(4)]
        + [pl.BlockSpec((8, F), lambda i: (0, 0)) for _ in range(4)]
        + [
            pl.BlockSpec((1, 1, PB), lambda i: (i, 0, 0)),
            pl.BlockSpec((NG, 1), lambda i: (0, 0)),
            pl.BlockSpec((1, NG), lambda i: (0, 0)),
            pl.BlockSpec((4 * F, F), lambda i: (0, 0)),
            pl.BlockSpec((1, F), lambda i: (0, 0)),
            pl.BlockSpec((F, OUT), lambda i: (0, 0)),
            pl.BlockSpec((1, OUT), lambda i: (0, 0)),
        ]
    ),
    out_specs=pl.BlockSpec((NS, OUT), lambda i: (0, 0)),
    out_shape=jax.ShapeDtypeStruct((NS, OUT), jnp.float32),
    scratch_shapes=[
        pltpu.VMEM((NG, 4 * F), jnp.float32),
        pltpu.VMEM((NG, 128), jnp.float32),
    ],
)


def kernel(x, edge_index, edge_attr, batch, weights, subgraph_batch, params):
    p = params
    src = edge_index[0]
    dst = edge_index[1]
    xpad = jnp.concatenate(
        [x, jnp.zeros((NPAD - N, F), jnp.float32)], axis=0)
    batchp = jnp.concatenate(
        [batch, jnp.full((NPAD - N,), NG, jnp.int32)])
    batch3 = batchp.reshape(_PBLK, 1, PB)
    sgb2 = subgraph_batch.reshape(1, NG)

    sc_gather = _make_sc_gather()
    sc_scatter = _make_sc_scatter()
    ab = jnp.concatenate(
        [jnp.ones((1, F), jnp.float32), jnp.zeros((7, F), jnp.float32)],
        axis=0)
    y = xpad
    ys, abs_ = [], []
    for i in (1, 2, 3, 4):
        hs = sc_gather(y, src)
        msgT = _msg_call(edge_attr, hs, ab,
                         p['c%d_beW1' % i], p['c%d_beb1' % i].reshape(1, F),
                         p['c%d_beW2' % i], p['c%d_beb2' % i].reshape(1, F))
        aggT = sc_scatter(msgT, dst)
        y, ab = _node_call(y, ab, aggT,
                           p['c%d_eps' % i].reshape(1, 1),
                           p['c%d_mW1' % i], p['c%d_mb1' % i].reshape(1, F),
                           p['c%d_mW2' % i], p['c%d_mb2' % i].reshape(1, F),
                           p['bn%d_g' % i].reshape(1, F),
                           p['bn%d_b' % i].reshape(1, F))
        ys.append(y)
        abs_.append(ab)

    return _pool_call(*ys, *abs_, batch3, weights, sgb2,
                      p['fc7_W'], p['fc7_b'].reshape(1, F),
                      p['pred_W'], p['pred_b'].reshape(1, OUT))


# speedup vs baseline: 2.1842x; 1.0010x over previous
import functools

import jax
import jax.numpy as jnp
from jax import lax
from jax.experimental import pallas as pl
from jax.experimental.pallas import tpu as pltpu
from jax.experimental.pallas import tpu_sc as plsc

N = 10000
NPAD = 10240
E = 160000
F = 256
NG = 512
NS = 64
OUT = 128

NCORE = 2
NSUB = 16
NWORK = NCORE * NSUB

GC = 200
EPW = E // NWORK


@functools.cache
def _make_sc_gather():
    @functools.partial(
        pl.kernel,
        out_type=jax.ShapeDtypeStruct((E, F), jnp.float32),
        mesh=plsc.VectorSubcoreMesh(core_axis_name="c", subcore_axis_name="s"),
        scratch_types=[
            pltpu.VMEM((GC,), jnp.int32),
            pltpu.VMEM((GC,), jnp.int32),
            pltpu.VMEM((GC, F), jnp.float32),
            pltpu.VMEM((GC, F), jnp.float32),
            pltpu.SemaphoreType.DMA,
            pltpu.SemaphoreType.DMA,
            pltpu.SemaphoreType.DMA,
            pltpu.SemaphoreType.DMA,
        ],
    )
    def _sc_gather(h_hbm, src_hbm, out_hbm, idx0, idx1, rows0, rows1,
                   si0, si1, sg0, sg1):
        wid = lax.axis_index("s") * NCORE + lax.axis_index("c")
        base0 = wid * EPW
        idxs = (idx0, idx1)
        rows = (rows0, rows1)
        sis = (si0, si1)
        sgs = (sg0, sg1)

        def istart(pp, t):
            pltpu.async_copy(src_hbm.at[pl.ds(base0 + t * GC, GC)],
                             idxs[pp], sis[pp])

        def iwait(pp, t):
            pltpu.make_async_copy(src_hbm.at[pl.ds(base0 + t * GC, GC)],
                                  idxs[pp], sis[pp]).wait()

        def gstart(pp):
            pltpu.async_copy(h_hbm.at[idxs[pp]], rows[pp], sgs[pp])

        def gwait(pp):
            pltpu.make_async_copy(h_hbm.at[idxs[pp]], rows[pp],
                                  sgs[pp]).wait()

        def wstart(pp, t):
            pltpu.async_copy(rows[pp], out_hbm.at[pl.ds(base0 + t * GC, GC)],
                             sgs[pp])

        def wwait(pp, t):
            pltpu.make_async_copy(rows[pp],
                                  out_hbm.at[pl.ds(base0 + t * GC, GC)],
                                  sgs[pp]).wait()

        istart(0, 0)
        iwait(0, 0)
        gstart(0)

        def pair(u, carry):
            t0 = 2 * u
            istart(1, t0 + 1)
            gwait(0)
            wstart(0, t0)
            iwait(1, t0 + 1)
            gstart(1)

            @pl.when(t0 + 2 < EPW // GC)
            def _():
                istart(0, t0 + 2)
                wwait(0, t0)
                iwait(0, t0 + 2)
                gstart(0)

            gwait(1)
            wstart(1, t0 + 1)
            wwait(1, t0 + 1)
            return carry

        lax.fori_loop(0, EPW // GC // 2, pair, 0)
        gwait(0)
        wstart(0, EPW // GC - 1)
        wwait(0, EPW // GC - 1)

    return _sc_gather


FS = 16
NH = NPAD // 2
ACC_C = NH + 128
DUMMY = NH + 64
SC_C = 1280
NCHUNK = E // SC_C


@functools.cache
def _make_sc_scatter():
    @functools.partial(
        pl.kernel,
        out_type=jax.ShapeDtypeStruct((F, NPAD), jnp.float32),
        mesh=plsc.VectorSubcoreMesh(core_axis_name="c", subcore_axis_name="s"),
        compiler_params=pltpu.CompilerParams(needs_layout_passes=False),
        scratch_types=[
            pltpu.VMEM((SC_C,), jnp.int32),
            pltpu.VMEM((SC_C,), jnp.int32),
            pltpu.VMEM((FS, SC_C), jnp.float32),
            pltpu.VMEM((FS, SC_C), jnp.float32),
            pltpu.SemaphoreType.DMA,
            pltpu.SemaphoreType.DMA,
            pltpu.VMEM((FS * ACC_C,), jnp.float32),
        ],
    )
    def _sc_scatter(msg_hbm, dst_hbm, out_hbm, dst0, dst1, buf0, buf1,
                    sem0, sem1, acc_v):
        c = lax.axis_index("c")
        s = lax.axis_index("s")
        base = c * NH
        iota = lax.iota(jnp.int32, 16)
        rot = [(iota + k) & 15 for k in range(16)]
        bufs = (buf0, buf1)
        dsts = (dst0, dst1)
        sems = (sem0, sem1)

        def start(pp, t):
            pltpu.async_copy(dst_hbm.at[pl.ds(t * SC_C, SC_C)], dsts[pp],
                             sems[pp])
            pltpu.async_copy(msg_hbm.at[s, t], bufs[pp], sems[pp])

        def wait(pp, t):
            pltpu.make_async_copy(dst_hbm.at[pl.ds(t * SC_C, SC_C)],
                                  dsts[pp], sems[pp]).wait()
            pltpu.make_async_copy(msg_hbm.at[s, t], bufs[pp],
                                  sems[pp]).wait()

        def compute(pp):
            buf_v = bufs[pp]
            dst_v = dsts[pp]

            @plsc.parallel_loop(0, SC_C // 16, unroll=4)
            def group(g):
                dvec = dst_v[pl.ds(g * 16, 16)]
                l = dvec - base
                ok = (l >= 0) & (l < NH)
                fbase = jnp.where(ok, l, DUMMY) * 16
                cidx = g * 16 + iota
                for k in range(16):
                    v = plsc.load_gather(buf_v, [rot[k], cidx])
                    plsc.addupdate_scatter(acc_v, [fbase + rot[k]], v)

        start(0, 0)

        @plsc.parallel_loop(0, FS * ACC_C // 16, unroll=4)
        def zrow(i):
            acc_v[pl.ds(i * 16, 16)] = jnp.zeros((16,), jnp.float32)

        def pair(u, carry):
            start(1, 2 * u + 1)
            wait(0, 2 * u)
            compute(0)
            start(0, 2 * u + 2)
            wait(1, 2 * u + 1)
            compute(1)
            return carry

        lax.fori_loop(0, (NCHUNK - 1) // 2, pair, 0)
        wait(0, NCHUNK - 1)
        compute(0)

        for q in range(NH // SC_C):
            @plsc.parallel_loop(0, SC_C // 16, unroll=2)
            def tgroup(g):
                nbase = (q * SC_C + g * 16) * 16 + iota * 16
                cidx = g * 16 + iota
                for k in range(16):
                    v = plsc.load_gather(acc_v, [nbase + rot[k]])
                    plsc.store_scatter(buf0, [rot[k], cidx], v)

            pltpu.sync_copy(
                buf0, out_hbm.at[pl.ds(FS * s, FS),
                                 pl.ds(base + q * SC_C, SC_C)])

    return _sc_scatter


BE = 1280


def _msg_body(ea_ref, hs_ref, ab_ref, w1_ref, b1_ref, w2_ref, b2_ref, o_ref):
    e = jnp.dot(ea_ref[...], w1_ref[...], preferred_element_type=jnp.float32)
    e = jnp.maximum(e + b1_ref[...], 0.0)
    e = jnp.dot(e, w2_ref[...], preferred_element_type=jnp.float32) + b2_ref[...]
    hs = hs_ref[...] * ab_ref[0:1, :] + ab_ref[1:2, :]
    m = jnp.maximum(e + hs, 0.0)
    o_ref[...] = jnp.transpose(m).reshape(FS, 1, FS, BE)


_msg_call = pl.pallas_call(
    _msg_body,
    grid=(E // BE,),
    in_specs=[
        pl.BlockSpec((BE, 16), lambda i: (i, 0)),
        pl.BlockSpec((BE, F), lambda i: (i, 0)),
        pl.BlockSpec((8, F), lambda i: (0, 0)),
        pl.BlockSpec((16, F), lambda i: (0, 0)),
        pl.BlockSpec((1, F), lambda i: (0, 0)),
        pl.BlockSpec((F, F), lambda i: (0, 0)),
        pl.BlockSpec((1, F), lambda i: (0, 0)),
    ],
    out_specs=pl.BlockSpec((FS, 1, FS, BE), lambda i: (0, i, 0, 0)),
    out_shape=jax.ShapeDtypeStruct((FS, E // BE, FS, BE), jnp.float32),
)

BN = 1280
_NBLK = NPAD // BN


def _node_body(y_ref, ab_ref, aggT_ref, eps_ref, w1_ref, b1_ref, w2_ref,
               b2_ref, g_ref, bb_ref, yo_ref, abo_ref, acc):
    i = pl.program_id(0)

    @pl.when(i == 0)
    def _():
        acc[...] = jnp.zeros_like(acc)

    x = y_ref[...] * ab_ref[0:1, :] + ab_ref[1:2, :]
    agg = jnp.transpose(aggT_ref[...])
    o = (1.0 + eps_ref[0, 0]) * x + agg
    t = jnp.dot(o, w1_ref[...], preferred_element_type=jnp.float32)
    t = jnp.maximum(t + b1_ref[...], 0.0)
    y = jnp.dot(t, w2_ref[...], preferred_element_type=jnp.float32)
    y = jnp.maximum(y + b2_ref[...], 0.0)
    yo_ref[...] = y
    gidx = i * BN + lax.broadcasted_iota(jnp.int32, (BN, 1), 0)
    ym = y * (gidx < N).astype(jnp.float32)
    acc[0:1, :] += jnp.sum(ym, axis=0, keepdims=True)
    acc[1:2, :] += jnp.sum(ym * ym, axis=0, keepdims=True)

    @pl.when(i == _NBLK - 1)
    def _():
        mu = acc[0:1, :] * (1.0 / N)
        var = acc[1:2, :] * (1.0 / N) - mu * mu
        al = g_ref[...] * lax.rsqrt(var + 1e-5)
        be = bb_ref[...] - mu * al
        abo_ref[...] = jnp.concatenate(
            [al, be, jnp.zeros((6, F), jnp.float32)], axis=0)


_node_call = pl.pallas_call(
    _node_body,
    grid=(_NBLK,),
    in_specs=[
        pl.BlockSpec((BN, F), lambda i: (i, 0)),
        pl.BlockSpec((8, F), lambda i: (0, 0)),
        pl.BlockSpec((F, BN), lambda i: (0, i)),
        pl.BlockSpec((1, 1), lambda i: (0, 0)),
        pl.BlockSpec((F, F), lambda i: (0, 0)),
        pl.BlockSpec((1, F), lambda i: (0, 0)),
        pl.BlockSpec((F, F), lambda i: (0, 0)),
        pl.BlockSpec((1, F), lambda i: (0, 0)),
        pl.BlockSpec((1, F), lambda i: (0, 0)),
        pl.BlockSpec((1, F), lambda i: (0, 0)),
    ],
    out_specs=[
        pl.BlockSpec((BN, F), lambda i: (i, 0)),
        pl.BlockSpec((8, F), lambda i: (0, 0)),
    ],
    out_shape=[
        jax.ShapeDtypeStruct((NPAD, F), jnp.float32),
        jax.ShapeDtypeStruct((8, F), jnp.float32),
    ],
    scratch_shapes=[pltpu.VMEM((8, F), jnp.float32)],
)

PB = 1280
_PBLK = NPAD // PB


def _pool_body(y1, y2, y3, y4, a1, a2, a3, a4, bt_ref, w_ref, sg_ref,
               f7w, f7b, pw, pb, out_ref, s_acc, c_acc):
    i = pl.program_id(0)

    @pl.when(i == 0)
    def _():
        s_acc[...] = jnp.zeros_like(s_acc)
        c_acc[...] = jnp.zeros_like(c_acc)

    bb = bt_ref[0, :, :]
    iota = lax.broadcasted_iota(jnp.int32, (NG, PB), 0)
    mask = (iota == bb).astype(jnp.float32)
    xc = jnp.concatenate([
        y1[...] * a1[0:1, :] + a1[1:2, :],
        y2[...] * a2[0:1, :] + a2[1:2, :],
        y3[...] * a3[0:1, :] + a3[1:2, :],
        y4[...] * a4[0:1, :] + a4[1:2, :],
    ], axis=1)
    s_acc[...] += jnp.dot(mask, xc, preferred_element_type=jnp.float32)
    c_acc[:, 0:1] += jnp.sum(mask, axis=1, keepdims=True)

    @pl.when(i == _PBLK - 1)
    def _():
        cnt = c_acc[:, 0:1]
        g = s_acc[...] / jnp.maximum(cnt, 1.0) * w_ref[...]
        sg = sg_ref[0:1, :]
        iota2 = lax.broadcasted_iota(jnp.int32, (NS, NG), 0)
        m2 = (iota2 == sg).astype(jnp.float32)
        gs = jnp.dot(m2, g, preferred_element_type=jnp.float32)
        norm = jnp.dot(m2, w_ref[...], preferred_element_type=jnp.float32)
        gs = gs / jnp.where(norm == 0.0, 1.0, norm)
        h = jnp.dot(gs, f7w[...], preferred_element_type=jnp.float32)
        h = jnp.maximum(h + f7b[...], 0.0)
        out_ref[...] = jnp.dot(h, pw[...],
                               preferred_element_type=jnp.float32) + pb[...]


_pool_call = pl.pallas_call(
    _pool_body,
    grid=(_PBLK,),
    in_specs=(
        [pl.BlockSpec((PB, F), lambda i: (i, 0)) for _ in range(4)]
        + [pl.BlockSpec((8, F), lambda i: (0, 0)) for _ in range(4)]
        + [
            pl.BlockSpec((1, 1, PB), lambda i: (i, 0, 0)),
            pl.BlockSpec((NG, 1), lambda i: (0, 0)),
            pl.BlockSpec((1, NG), lambda i: (0, 0)),
            pl.BlockSpec((4 * F, F), lambda i: (0, 0)),
            pl.BlockSpec((1, F), lambda i: (0, 0)),
            pl.BlockSpec((F, OUT), lambda i: (0, 0)),
            pl.BlockSpec((1, OUT), lambda i: (0, 0)),
        ]
    ),
    out_specs=pl.BlockSpec((NS, OUT), lambda i: (0, 0)),
    out_shape=jax.ShapeDtypeStruct((NS, OUT), jnp.float32),
    scratch_shapes=[
        pltpu.VMEM((NG, 4 * F), jnp.float32),
        pltpu.VMEM((NG, 128), jnp.float32),
    ],
)


def kernel(x, edge_index, edge_attr, batch, weights, subgraph_batch, params):
    p = params
    src = edge_index[0]
    dst = edge_index[1]
    xpad = jnp.concatenate(
        [x, jnp.zeros((NPAD - N, F), jnp.float32)], axis=0)
    batchp = jnp.concatenate(
        [batch, jnp.full((NPAD - N,), NG, jnp.int32)])
    batch3 = batchp.reshape(_PBLK, 1, PB)
    sgb2 = subgraph_batch.reshape(1, NG)

    sc_gather = _make_sc_gather()
    sc_scatter = _make_sc_scatter()
    ab = jnp.concatenate(
        [jnp.ones((1, F), jnp.float32), jnp.zeros((7, F), jnp.float32)],
        axis=0)
    y = xpad
    ys, abs_ = [], []
    for i in (1, 2, 3, 4):
        hs = sc_gather(y, src)
        msgT = _msg_call(edge_attr, hs, ab,
                         p['c%d_beW1' % i], p['c%d_beb1' % i].reshape(1, F),
                         p['c%d_beW2' % i], p['c%d_beb2' % i].reshape(1, F))
        aggT = sc_scatter(msgT, dst)
        y, ab = _node_call(y, ab, aggT,
                           p['c%d_eps' % i].reshape(1, 1),
                           p['c%d_mW1' % i], p['c%d_mb1' % i].reshape(1, F),
                           p['c%d_mW2' % i], p['c%d_mb2' % i].reshape(1, F),
                           p['bn%d_g' % i].reshape(1, F),
                           p['bn%d_b' % i].reshape(1, F))
        ys.append(y)
        abs_.append(ab)

    return _pool_call(*ys, *abs_, batch3, weights, sgb2,
                      p['fc7_W'], p['fc7_b'].reshape(1, F),
                      p['pred_W'], p['pred_b'].reshape(1, OUT))
